# Initial kernel scaffold; baseline (speedup 1.0000x reference)
import jax
import jax.numpy as jnp
from jax.experimental import pallas as pl


def kernel(vert, edge, W, a_l, a_r):
    raise NotImplementedError("write your pallas kernel here")


# speedup vs baseline: 1.8750x; 1.8750x over previous
import jax
import jax.numpy as jnp
from jax.experimental import pallas as pl
from jax.experimental.pallas import tpu as pltpu

_N = 4096
_IN_F = 128
_HEADS = 2
_HID = 32
_BI = 256


def _gat_body(vert_ref, edge_ref, w_ref, al_ref, ar_ref, out_ref, g_ref, srt_ref):
    i = pl.program_id(0)

    @pl.when(i == 0)
    def _init():
        g = jnp.dot(vert_ref[...], w_ref[...], preferred_element_type=jnp.float32)
        for h in range(_HEADS):
            gh = g[:, h * _HID:(h + 1) * _HID]
            g_ref[h] = gh
            srt_ref[h:h + 1, :] = jax.lax.dot_general(
                ar_ref[...], gh, (((1,), (1,)), ((), ())),
                preferred_element_type=jnp.float32)

    mask = edge_ref[...] != 0
    for h in range(_HEADS):
        gh = g_ref[h]
        g_blk = g_ref[h, pl.ds(i * _BI, _BI), :]
        sl = jax.lax.dot_general(
            g_blk, al_ref[...], (((1,), (1,)), ((), ())),
            preferred_element_type=jnp.float32)
        s = sl + srt_ref[h:h + 1, :]
        s = jnp.where(s >= 0, s, 0.2 * s)
        s = jnp.where(mask, s, jnp.float32(-1e9))
        m = jnp.max(s, axis=1, keepdims=True)
        p = jnp.exp(s - m)
        d = jnp.sum(p, axis=1, keepdims=True)
        o = jnp.dot(p, gh, preferred_element_type=jnp.float32) / d
        out_ref[:, h * _HID:(h + 1) * _HID] = jnp.where(o > 0, o, jnp.exp(o) - 1.0)


def kernel(vert, edge, W, a_l, a_r):
    edge_i8 = edge.astype(jnp.int8)
    al2 = a_l.reshape(1, _HID)
    ar2 = a_r.reshape(1, _HID)
    return pl.pallas_call(
        _gat_body,
        grid=(_N // _BI,),
        in_specs=[
            pl.BlockSpec((_N, _IN_F), lambda i: (0, 0)),
            pl.BlockSpec((_BI, _N), lambda i: (i, 0)),
            pl.BlockSpec((_IN_F, _HEADS * _HID), lambda i: (0, 0)),
            pl.BlockSpec((1, _HID), lambda i: (0, 0)),
            pl.BlockSpec((1, _HID), lambda i: (0, 0)),
        ],
        out_specs=pl.BlockSpec((_BI, _HEADS * _HID), lambda i: (i, 0)),
        out_shape=jax.ShapeDtypeStruct((_N, _HEADS * _HID), jnp.float32),
        scratch_shapes=[
            pltpu.VMEM((_HEADS, _N, _HID), jnp.float32),
            pltpu.VMEM((_HEADS, _N), jnp.float32),
        ],
    )(vert, edge_i8, W, al2, ar2)


# speedup vs baseline: 3.3831x; 1.8043x over previous
import jax
import jax.numpy as jnp
from jax.experimental import pallas as pl
from jax.experimental.pallas import tpu as pltpu

_N = 4096
_IN_F = 128
_HEADS = 2
_HID = 32
_BI = 256


def _gat_body(vert_ref, edge_ref, w_ref, al_ref, ar_ref, out_ref,
              ghe_ref, f1_ref, f2_ref, srt_ref, smax_ref, gsum_ref):
    i = pl.program_id(0)

    @pl.when(i == 0)
    def _init():
        g = jnp.dot(vert_ref[...], w_ref[...], preferred_element_type=jnp.float32)
        for h in range(_HEADS):
            gh = g[:, h * _HID:(h + 1) * _HID]
            ghe_ref[h, :, 0:_HID] = gh
            ghe_ref[h, :, _HID:_HID + 1] = jnp.ones((_N, 1), jnp.float32)
            sr = jax.lax.dot_general(
                ar_ref[...], gh, (((1,), (1,)), ((), ())),
                preferred_element_type=jnp.float32)
            s_max = jnp.max(sr)
            smax_ref[h] = s_max
            srt_ref[h:h + 1, :] = sr
            f1_ref[h:h + 1, :] = jnp.exp(sr - s_max)
            f2_ref[h:h + 1, :] = jnp.exp(0.2 * (sr - s_max))
            gsum_ref[h:h + 1, :] = jnp.sum(gh, axis=0, keepdims=True) * (1.0 / _N)

    maskf = edge_ref[...].astype(jnp.float32)
    for h in range(_HEADS):
        g_blk = ghe_ref[h, pl.ds(i * _BI, _BI), 0:_HID]
        sl = jax.lax.dot_general(
            g_blk, al_ref[...], (((1,), (1,)), ((), ())),
            preferred_element_type=jnp.float32)
        s_max = smax_ref[h]
        m = jnp.maximum(sl, -s_max)
        e1 = jnp.exp(sl - m)
        e2 = jnp.exp(0.2 * sl - m - 0.8 * s_max)
        cond = srt_ref[h:h + 1, :] >= -sl
        p = jnp.where(cond, e1 * f1_ref[h:h + 1, :], e2 * f2_ref[h:h + 1, :]) * maskf
        o_ext = jnp.dot(p, ghe_ref[h], preferred_element_type=jnp.float32)
        o = o_ext[:, 0:_HID]
        d = o_ext[:, _HID:_HID + 1]
        o = jnp.where(d > 0, o / d, gsum_ref[h:h + 1, :])
        out_ref[:, h * _HID:(h + 1) * _HID] = jnp.where(o > 0, o, jnp.exp(o) - 1.0)


def kernel(vert, edge, W, a_l, a_r):
    edge_i8 = edge.astype(jnp.int8)
    al2 = a_l.reshape(1, _HID)
    ar2 = a_r.reshape(1, _HID)
    return pl.pallas_call(
        _gat_body,
        grid=(_N // _BI,),
        in_specs=[
            pl.BlockSpec((_N, _IN_F), lambda i: (0, 0)),
            pl.BlockSpec((_BI, _N), lambda i: (i, 0)),
            pl.BlockSpec((_IN_F, _HEADS * _HID), lambda i: (0, 0)),
            pl.BlockSpec((1, _HID), lambda i: (0, 0)),
            pl.BlockSpec((1, _HID), lambda i: (0, 0)),
        ],
        out_specs=pl.BlockSpec((_BI, _HEADS * _HID), lambda i: (i, 0)),
        out_shape=jax.ShapeDtypeStruct((_N, _HEADS * _HID), jnp.float32),
        scratch_shapes=[
            pltpu.VMEM((_HEADS, _N, _HID + 1), jnp.float32),
            pltpu.VMEM((_HEADS, _N), jnp.float32),
            pltpu.VMEM((_HEADS, _N), jnp.float32),
            pltpu.VMEM((_HEADS, _N), jnp.float32),
            pltpu.SMEM((_HEADS,), jnp.float32),
            pltpu.VMEM((_HEADS, _HID), jnp.float32),
        ],
    )(vert, edge_i8, W, al2, ar2)


# speedup vs baseline: 3.6724x; 1.0855x over previous
import jax
import jax.numpy as jnp
from jax.experimental import pallas as pl
from jax.experimental.pallas import tpu as pltpu

_N = 4096
_IN_F = 128
_HEADS = 2
_HID = 32
_BI = 256


def _gat_body(vert_ref, edge_ref, w_ref, al_ref, ar_ref, out_ref,
              ghe_ref, f1_ref, f2_ref, smax_ref, gsum_ref):
    i = pl.program_id(0)

    @pl.when(i == 0)
    def _init():
        g = jnp.dot(vert_ref[...], w_ref[...], preferred_element_type=jnp.float32)
        for h in range(_HEADS):
            gh = g[:, h * _HID:(h + 1) * _HID]
            ghe_ref[h, :, 0:_HID] = gh
            ghe_ref[h, :, _HID:_HID + 1] = jnp.ones((_N, 1), jnp.float32)
            sr = jax.lax.dot_general(
                ar_ref[...], gh, (((1,), (1,)), ((), ())),
                preferred_element_type=jnp.float32)
            s_max = jnp.max(sr)
            smax_ref[h] = s_max
            f1_ref[h:h + 1, :] = jnp.exp(sr - s_max)
            f2_ref[h:h + 1, :] = jnp.exp(0.2 * (sr - s_max))
            gsum_ref[h:h + 1, :] = jnp.sum(gh, axis=0, keepdims=True) * (1.0 / _N)

    maskf = edge_ref[...].astype(jnp.float32)
    for h in range(_HEADS):
        g_blk = ghe_ref[h, pl.ds(i * _BI, _BI), 0:_HID]
        sl = jax.lax.dot_general(
            g_blk, al_ref[...], (((1,), (1,)), ((), ())),
            preferred_element_type=jnp.float32)
        s_max = smax_ref[h]
        m = jnp.maximum(sl, -s_max)
        e1 = jnp.exp(sl - m)
        e2 = jnp.exp(0.2 * sl - m - 0.8 * s_max)
        p = jnp.maximum(e1 * f1_ref[h:h + 1, :], e2 * f2_ref[h:h + 1, :]) * maskf
        o_ext = jnp.dot(p, ghe_ref[h], preferred_element_type=jnp.float32)
        o = o_ext[:, 0:_HID]
        d = o_ext[:, _HID:_HID + 1]
        o = jnp.where(d > 0, o / d, gsum_ref[h:h + 1, :])
        out_ref[:, h * _HID:(h + 1) * _HID] = jnp.where(o > 0, o, jnp.exp(o) - 1.0)


def kernel(vert, edge, W, a_l, a_r):
    edge_i8 = edge.astype(jnp.int8)
    al2 = a_l.reshape(1, _HID)
    ar2 = a_r.reshape(1, _HID)
    return pl.pallas_call(
        _gat_body,
        grid=(_N // _BI,),
        in_specs=[
            pl.BlockSpec((_N, _IN_F), lambda i: (0, 0)),
            pl.BlockSpec((_BI, _N), lambda i: (i, 0)),
            pl.BlockSpec((_IN_F, _HEADS * _HID), lambda i: (0, 0)),
            pl.BlockSpec((1, _HID), lambda i: (0, 0)),
            pl.BlockSpec((1, _HID), lambda i: (0, 0)),
        ],
        out_specs=pl.BlockSpec((_BI, _HEADS * _HID), lambda i: (i, 0)),
        out_shape=jax.ShapeDtypeStruct((_N, _HEADS * _HID), jnp.float32),
        scratch_shapes=[
            pltpu.VMEM((_HEADS, _N, _HID + 1), jnp.float32),
            pltpu.VMEM((_HEADS, _N), jnp.float32),
            pltpu.VMEM((_HEADS, _N), jnp.float32),
            pltpu.SMEM((_HEADS,), jnp.float32),
            pltpu.VMEM((_HEADS, _HID), jnp.float32),
        ],
    )(vert, edge_i8, W, al2, ar2)


# speedup vs baseline: 3.9875x; 1.0858x over previous
import jax
import jax.numpy as jnp
from jax.experimental import pallas as pl
from jax.experimental.pallas import tpu as pltpu

_N = 4096
_IN_F = 128
_HEADS = 2
_HID = 32
_BI = 256


def _gat_body(vert_ref, edge_ref, w_ref, al_ref, ar_ref, out_ref,
              ghe_ref, f1_ref, f2_ref, smax_ref, gsum_ref):
    i = pl.program_id(0)

    @pl.when(i == 0)
    def _init():
        g = jnp.dot(vert_ref[...], w_ref[...], preferred_element_type=jnp.float32)
        for h in range(_HEADS):
            gh = g[:, h * _HID:(h + 1) * _HID]
            ghe_ref[h, :, 0:_HID] = gh.astype(jnp.bfloat16)
            ghe_ref[h, :, _HID:_HID + 1] = jnp.ones((_N, 1), jnp.bfloat16)
            sr = jax.lax.dot_general(
                ar_ref[...], gh, (((1,), (1,)), ((), ())),
                preferred_element_type=jnp.float32)
            s_max = jnp.max(sr)
            smax_ref[h] = s_max
            f1_ref[h:h + 1, :] = jnp.exp(sr - s_max).astype(jnp.bfloat16)
            f2_ref[h:h + 1, :] = jnp.exp(0.2 * (sr - s_max)).astype(jnp.bfloat16)
            gsum_ref[h:h + 1, :] = jnp.sum(gh, axis=0, keepdims=True) * (1.0 / _N)

    maskf = edge_ref[...].astype(jnp.bfloat16)
    for h in range(_HEADS):
        g_blk = ghe_ref[h, pl.ds(i * _BI, _BI), 0:_HID].astype(jnp.float32)
        sl = jax.lax.dot_general(
            g_blk, al_ref[...], (((1,), (1,)), ((), ())),
            preferred_element_type=jnp.float32)
        s_max = smax_ref[h]
        m = jnp.maximum(sl, -s_max)
        e1 = jnp.exp(sl - m).astype(jnp.bfloat16)
        e2 = jnp.exp(0.2 * sl - m - 0.8 * s_max).astype(jnp.bfloat16)
        p = jnp.maximum(e1 * f1_ref[h:h + 1, :], e2 * f2_ref[h:h + 1, :]) * maskf
        o_ext = jnp.dot(p, ghe_ref[h], preferred_element_type=jnp.float32)
        o = o_ext[:, 0:_HID]
        d = o_ext[:, _HID:_HID + 1]
        o = jnp.where(d > 0, o / d, gsum_ref[h:h + 1, :])
        out_ref[:, h * _HID:(h + 1) * _HID] = jnp.where(o > 0, o, jnp.exp(o) - 1.0)


def kernel(vert, edge, W, a_l, a_r):
    edge_i8 = edge.astype(jnp.int8)
    al2 = a_l.reshape(1, _HID)
    ar2 = a_r.reshape(1, _HID)
    return pl.pallas_call(
        _gat_body,
        grid=(_N // _BI,),
        in_specs=[
            pl.BlockSpec((_N, _IN_F), lambda i: (0, 0)),
            pl.BlockSpec((_BI, _N), lambda i: (i, 0)),
            pl.BlockSpec((_IN_F, _HEADS * _HID), lambda i: (0, 0)),
            pl.BlockSpec((1, _HID), lambda i: (0, 0)),
            pl.BlockSpec((1, _HID), lambda i: (0, 0)),
        ],
        out_specs=pl.BlockSpec((_BI, _HEADS * _HID), lambda i: (i, 0)),
        out_shape=jax.ShapeDtypeStruct((_N, _HEADS * _HID), jnp.float32),
        scratch_shapes=[
            pltpu.VMEM((_HEADS, _N, _HID + 1), jnp.bfloat16),
            pltpu.VMEM((_HEADS, _N), jnp.bfloat16),
            pltpu.VMEM((_HEADS, _N), jnp.bfloat16),
            pltpu.SMEM((_HEADS,), jnp.float32),
            pltpu.VMEM((_HEADS, _HID), jnp.float32),
        ],
    )(vert, edge_i8, W, al2, ar2)
